# Initial kernel scaffold; baseline (speedup 1.0000x reference)
import jax
import jax.numpy as jnp
from jax.experimental import pallas as pl


def kernel(away_indices, home_indices, away_stats, home_stats, away_weights, home_weights, W_stat, b_stat, emb, W1, b1, W2, b2, W3, b3):
    raise NotImplementedError("write your pallas kernel here")


# speedup vs baseline: 2.1131x; 2.1131x over previous
import functools

import jax
import jax.numpy as jnp
from jax import lax
from jax.experimental import pallas as pl
from jax.experimental.pallas import tpu as pltpu
from jax.experimental.pallas import tpu_sc as plsc

B = 16384
K = 20
S = 5
D = 32
H = 64

NC = 2
NS = 16
NW = NC * NS
ROWS = 2 * B
RPW = ROWS // NW
CH = 32
G = CH * K
NCHUNK = RPW // CH
GSUB = 128
NSUB = G // GSUB


def _sc_weighted_embed(emb, idx_flat, w_flat):
  mesh = plsc.VectorSubcoreMesh(core_axis_name="c", subcore_axis_name="s")

  @functools.partial(
      pl.kernel,
      mesh=mesh,
      compiler_params=pltpu.CompilerParams(use_tc_tiling_on_sc=False),
      out_type=jax.ShapeDtypeStruct((ROWS, D), jnp.float32),
      scratch_types=[
          pltpu.VMEM((G,), jnp.int32),
          pltpu.VMEM((G + 16,), jnp.float32),
          pltpu.VMEM((G, D), jnp.float32),
          pltpu.VMEM((CH, D), jnp.float32),
          pltpu.SemaphoreType.DMA,
      ],
  )
  def body(emb_hbm, idx_hbm, w_hbm, out_hbm, idx_v, w_v, rows_v, acc_v, sem):
    wid = lax.axis_index("s") * NC + lax.axis_index("c")
    base = wid * RPW

    def chunk_body(c, carry):
      row0 = base + c * CH
      g0 = row0 * K
      pltpu.sync_copy(idx_hbm.at[pl.ds(g0, G)], idx_v)
      pltpu.sync_copy(w_hbm.at[pl.ds(g0, G)], w_v.at[pl.ds(0, G)])
      cps = [
          pltpu.async_copy(
              emb_hbm.at[idx_v.at[pl.ds(j * GSUB, GSUB)]],
              rows_v.at[pl.ds(j * GSUB, GSUB)],
              sem,
          )
          for j in range(NSUB)
      ]
      for cp in cps:
        cp.wait()

      def row_body(b, carry2):
        g = b * K
        wvA = w_v[pl.ds(g, 16)]
        wvB = w_v[pl.ds(g + 8, 16)]
        wk0 = wvA[0]
        acc0 = rows_v[g, pl.ds(0, 16)] * wk0
        acc1 = rows_v[g, pl.ds(16, 16)] * wk0
        for k in range(1, K):
          wk = wvA[k] if k < 16 else wvB[k - 8]
          acc0 = acc0 + rows_v[g + k, pl.ds(0, 16)] * wk
          acc1 = acc1 + rows_v[g + k, pl.ds(16, 16)] * wk
        acc_v[b, pl.ds(0, 16)] = acc0
        acc_v[b, pl.ds(16, 16)] = acc1
        return carry2

      lax.fori_loop(0, CH, row_body, 0)
      pltpu.sync_copy(acc_v, out_hbm.at[pl.ds(row0, CH)])
      return carry

    lax.fori_loop(0, NCHUNK, chunk_body, 0)

  return body(emb, idx_flat, w_flat)


BS = 2048
GRID = B // BS


def _tc_head(a_s2, a_we, h_s2, h_we, e_all, W_big, b_stat2, W1a, W1b, b1_2,
             W2, b2_2, W3, b3_2):
  def body(a_s, a_w, h_s, h_w, ea, eh, wb, bst, w1a, w1b, bb1, w2, bb2, w3,
           bb3, out):
    f32 = jnp.float32
    dot = lambda x, y: lax.dot_general(x, y, (((1,), (0,)), ((), ())),
                                       preferred_element_type=f32)
    pa = dot(a_s[...] * a_w[...], wb[...]) + ea[...]
    pa = pa + (jnp.sum(a_w[...], axis=1, keepdims=True) * (1.0 / S)) * bst[...]
    ph = dot(h_s[...] * h_w[...], wb[...]) + eh[...]
    ph = ph + (jnp.sum(h_w[...], axis=1, keepdims=True) * (1.0 / S)) * bst[...]
    h1 = jnp.maximum(dot(pa, w1a[...]) + dot(ph, w1b[...]) + bb1[...], 0.0)
    h2 = jnp.maximum(dot(h1, w2[...]) + bb2[...], 0.0)
    out[...] = dot(h2, w3[...]) + bb3[...]

  KS = K * S
  in_specs = [
      pl.BlockSpec((BS, KS), lambda i: (i, 0)),
      pl.BlockSpec((BS, KS), lambda i: (i, 0)),
      pl.BlockSpec((BS, KS), lambda i: (i, 0)),
      pl.BlockSpec((BS, KS), lambda i: (i, 0)),
      pl.BlockSpec((BS, D), lambda i: (i, 0)),
      pl.BlockSpec((BS, D), lambda i: (i + GRID, 0)),
      pl.BlockSpec((KS, D), lambda i: (0, 0)),
      pl.BlockSpec((1, D), lambda i: (0, 0)),
      pl.BlockSpec((D, H), lambda i: (0, 0)),
      pl.BlockSpec((D, H), lambda i: (0, 0)),
      pl.BlockSpec((1, H), lambda i: (0, 0)),
      pl.BlockSpec((H, H), lambda i: (0, 0)),
      pl.BlockSpec((1, H), lambda i: (0, 0)),
      pl.BlockSpec((H, 1), lambda i: (0, 0)),
      pl.BlockSpec((1, 1), lambda i: (0, 0)),
  ]
  return pl.pallas_call(
      body,
      grid=(GRID,),
      in_specs=in_specs,
      out_specs=pl.BlockSpec((BS, 1), lambda i: (i, 0)),
      out_shape=jax.ShapeDtypeStruct((B, 1), jnp.float32),
  )(a_s2, a_we, h_s2, h_we, e_all, e_all, W_big, b_stat2, W1a, W1b, b1_2,
    W2, b2_2, W3, b3_2)


def kernel(away_indices, home_indices, away_stats, home_stats, away_weights,
           home_weights, W_stat, b_stat, emb, W1, b1, W2, b2, W3, b3):
  idx_flat = jnp.concatenate(
      [away_indices.reshape(-1), home_indices.reshape(-1)]).astype(jnp.int32)
  w_flat = jnp.concatenate(
      [away_weights.reshape(-1), home_weights.reshape(-1)])
  e_all = _sc_weighted_embed(emb, idx_flat, w_flat)

  a_s2 = away_stats.reshape(B, K * S)
  h_s2 = home_stats.reshape(B, K * S)
  a_we = jnp.repeat(away_weights, S, axis=1)
  h_we = jnp.repeat(home_weights, S, axis=1)
  W_big = jnp.tile(W_stat, (K, 1))
  out = _tc_head(a_s2, a_we, h_s2, h_we, e_all, W_big, b_stat.reshape(1, D),
                 W1[:D], W1[D:], b1.reshape(1, H), W2, b2.reshape(1, H),
                 W3, b3.reshape(1, 1))
  return out[:, 0]
